# baseline (speedup 1.0000x reference)
import jax
import jax.numpy as jnp
from jax.experimental import pallas as pl
from jax.experimental.pallas import tpu as pltpu

TOPK = 50
NROWS = 128
NCOLS = 32768
BLOCK_ROWS = 64


def _topk_mask_body(x_ref, o_ref):
    x = x_ref[...]
    u = jax.lax.bitcast_convert_type(x, jnp.uint32)
    sign = u >= jnp.uint32(0x80000000)
    key = jnp.where(sign, ~u, u | jnp.uint32(0x80000000))

    prefix = jnp.zeros((BLOCK_ROWS, 1), jnp.uint32)
    for b in range(31, -1, -1):
        cand = prefix | jnp.uint32(1 << b)
        cnt = jnp.sum((key >= cand).astype(jnp.int32), axis=1, keepdims=True)
        prefix = jnp.where(cnt >= TOPK, cand, prefix)
    kth = prefix

    greater = key > kth
    eq = key == kth
    n_greater = jnp.sum(greater.astype(jnp.int32), axis=1, keepdims=True)
    m = jnp.sum(eq.astype(jnp.int32), axis=1, keepdims=True)
    r = TOPK - n_greater

    col = jax.lax.broadcasted_iota(jnp.int32, (BLOCK_ROWS, NCOLS), 1)

    def tie_search(_):
        v = jnp.where(eq, col, -1)
        want = m - r + 1
        iprefix = jnp.zeros((BLOCK_ROWS, 1), jnp.int32)
        for b in range(14, -1, -1):
            cand = iprefix | jnp.int32(1 << b)
            cnt = jnp.sum((v >= cand).astype(jnp.int32), axis=1,
                          keepdims=True)
            iprefix = jnp.where(cnt >= want, cand, iprefix)
        return jnp.where(r == m, jnp.int32(NCOLS - 1), iprefix)

    def no_tie(_):
        return jnp.full((BLOCK_ROWS, 1), NCOLS - 1, jnp.int32)

    ithresh = jax.lax.cond(jnp.any(r < m), tie_search, no_tie, 0)

    mask = greater | (eq & (col <= ithresh))
    o_ref[...] = jnp.where(mask, x, 0.0)


def kernel(x):
    return pl.pallas_call(
        _topk_mask_body,
        grid=(NROWS // BLOCK_ROWS,),
        in_specs=[pl.BlockSpec((BLOCK_ROWS, NCOLS), lambda i: (i, 0))],
        out_specs=pl.BlockSpec((BLOCK_ROWS, NCOLS), lambda i: (i, 0)),
        out_shape=jax.ShapeDtypeStruct((NROWS, NCOLS), jnp.float32),
    )(x)


# speedup vs baseline: 1.3070x; 1.3070x over previous
import jax
import jax.numpy as jnp
from jax.experimental import pallas as pl
from jax.experimental.pallas import tpu as pltpu

TOPK = 50
NROWS = 128
NCOLS = 32768
BLOCK_ROWS = 16


def _topk_mask_body(x_ref, o_ref):
    x = x_ref[...]
    u = jax.lax.bitcast_convert_type(x, jnp.uint32)
    sign = u >= jnp.uint32(0x80000000)
    key = jnp.where(sign, ~u, u | jnp.uint32(0x80000000))

    prefix = jnp.zeros((BLOCK_ROWS, 1), jnp.uint32)
    for b in range(31, -1, -1):
        cand = prefix | jnp.uint32(1 << b)
        cnt = jnp.sum((key >= cand).astype(jnp.int32), axis=1, keepdims=True)
        prefix = jnp.where(cnt >= TOPK, cand, prefix)
    kth = prefix

    greater = key > kth
    eq = key == kth
    n_greater = jnp.sum(greater.astype(jnp.int32), axis=1, keepdims=True)
    m = jnp.sum(eq.astype(jnp.int32), axis=1, keepdims=True)
    r = TOPK - n_greater

    col = jax.lax.broadcasted_iota(jnp.int32, (BLOCK_ROWS, NCOLS), 1)

    def tie_search(_):
        v = jnp.where(eq, col, -1)
        want = m - r + 1
        iprefix = jnp.zeros((BLOCK_ROWS, 1), jnp.int32)
        for b in range(14, -1, -1):
            cand = iprefix | jnp.int32(1 << b)
            cnt = jnp.sum((v >= cand).astype(jnp.int32), axis=1,
                          keepdims=True)
            iprefix = jnp.where(cnt >= want, cand, iprefix)
        return jnp.where(r == m, jnp.int32(NCOLS - 1), iprefix)

    def no_tie(_):
        return jnp.full((BLOCK_ROWS, 1), NCOLS - 1, jnp.int32)

    ithresh = jax.lax.cond(jnp.any(r < m), tie_search, no_tie, 0)

    mask = greater | (eq & (col <= ithresh))
    o_ref[...] = jnp.where(mask, x, 0.0)


def kernel(x):
    return pl.pallas_call(
        _topk_mask_body,
        grid=(NROWS // BLOCK_ROWS,),
        in_specs=[pl.BlockSpec((BLOCK_ROWS, NCOLS), lambda i: (i, 0))],
        out_specs=pl.BlockSpec((BLOCK_ROWS, NCOLS), lambda i: (i, 0)),
        out_shape=jax.ShapeDtypeStruct((NROWS, NCOLS), jnp.float32),
    )(x)


# speedup vs baseline: 1.3903x; 1.0638x over previous
import jax
import jax.numpy as jnp
from jax.experimental import pallas as pl
from jax.experimental.pallas import tpu as pltpu

TOPK = 50
NROWS = 128
NCOLS = 32768
BLOCK_ROWS = 32


def _topk_mask_body(x_ref, o_ref):
    x = x_ref[...]
    u = jax.lax.bitcast_convert_type(x, jnp.uint32)
    sign = u >= jnp.uint32(0x80000000)
    key = jnp.where(sign, ~u, u | jnp.uint32(0x80000000))

    prefix = jnp.zeros((BLOCK_ROWS, 1), jnp.uint32)
    for b in range(31, -1, -1):
        cand = prefix | jnp.uint32(1 << b)
        cnt = jnp.sum((key >= cand).astype(jnp.int32), axis=1, keepdims=True)
        prefix = jnp.where(cnt >= TOPK, cand, prefix)
    kth = prefix

    greater = key > kth
    eq = key == kth
    n_greater = jnp.sum(greater.astype(jnp.int32), axis=1, keepdims=True)
    m = jnp.sum(eq.astype(jnp.int32), axis=1, keepdims=True)
    r = TOPK - n_greater

    col = jax.lax.broadcasted_iota(jnp.int32, (BLOCK_ROWS, NCOLS), 1)

    def tie_search(_):
        v = jnp.where(eq, col, -1)
        want = m - r + 1
        iprefix = jnp.zeros((BLOCK_ROWS, 1), jnp.int32)
        for b in range(14, -1, -1):
            cand = iprefix | jnp.int32(1 << b)
            cnt = jnp.sum((v >= cand).astype(jnp.int32), axis=1,
                          keepdims=True)
            iprefix = jnp.where(cnt >= want, cand, iprefix)
        return jnp.where(r == m, jnp.int32(NCOLS - 1), iprefix)

    def no_tie(_):
        return jnp.full((BLOCK_ROWS, 1), NCOLS - 1, jnp.int32)

    ithresh = jax.lax.cond(jnp.any(r < m), tie_search, no_tie, 0)

    mask = greater | (eq & (col <= ithresh))
    o_ref[...] = jnp.where(mask, x, 0.0)


def kernel(x):
    return pl.pallas_call(
        _topk_mask_body,
        grid=(NROWS // BLOCK_ROWS,),
        in_specs=[pl.BlockSpec((BLOCK_ROWS, NCOLS), lambda i: (i, 0))],
        out_specs=pl.BlockSpec((BLOCK_ROWS, NCOLS), lambda i: (i, 0)),
        out_shape=jax.ShapeDtypeStruct((NROWS, NCOLS), jnp.float32),
    )(x)
